# Initial kernel scaffold; baseline (speedup 1.0000x reference)
import jax
import jax.numpy as jnp
from jax.experimental import pallas as pl


def kernel(real, imag, graph, W1, b1, W2, b2, Wlin, blin, Wconv, bconv):
    raise NotImplementedError("write your pallas kernel here")


# speedup vs baseline: 61.6171x; 61.6171x over previous
import functools

import jax
import jax.numpy as jnp
from jax.experimental import pallas as pl

B, N, IN_C, F_, T, NC = 1024, 30, 128, 64, 16, 4
BB = 8


def _gcn_body(graph_ref, real_ref, W1_ref, b1_ref, W2_ref, b2_ref,
              Wlin_ref, blin_ref, WconvT_ref, bconv_ref, out_ref):
    g = graph_ref[...]
    adj = jnp.sum(g, axis=1) * (1.0 / T)
    w = (adj != 0.0).astype(jnp.float32)
    rr = jax.lax.broadcasted_iota(jnp.int32, (N, N), 0)
    cc = jax.lax.broadcasted_iota(jnp.int32, (N, N), 1)
    eye = (rr == cc).astype(jnp.float32)
    A = w + eye[None]
    deg = jnp.sum(A, axis=1)
    dinv = jax.lax.rsqrt(deg)
    M = dinv[:, :, None] * A * dinv[:, None, :]

    W1 = W1_ref[...]
    W2 = W2_ref[...]
    b1 = b1_ref[...]
    b2 = b2_ref[...]

    h2s = []
    for i in range(BB):
        MT = M[i].T
        h = jnp.dot(real_ref[i], W1,
                    preferred_element_type=jnp.float32)
        h1 = jnp.maximum(
            jnp.dot(MT, h, preferred_element_type=jnp.float32) + b1, 0.0)
        g2 = jnp.dot(h1, W2, preferred_element_type=jnp.float32)
        h2 = jnp.maximum(
            jnp.dot(MT, g2, preferred_element_type=jnp.float32) + b2, 0.0)
        h2s.append(h2[None])
    h2a = jnp.concatenate(h2s, axis=0)

    lin = jnp.sum(h2a * Wlin_ref[...][None], axis=2)
    xl = jnp.maximum(lin + blin_ref[0, 0], 0.0)
    out = jnp.dot(xl, WconvT_ref[...],
                  preferred_element_type=jnp.float32) + bconv_ref[...]
    out_ref[...] = out


@jax.jit
def kernel(real, imag, graph, W1, b1, W2, b2, Wlin, blin, Wconv, bconv):
    del imag
    grid = (B // BB,)
    out = pl.pallas_call(
        _gcn_body,
        grid=grid,
        in_specs=[
            pl.BlockSpec((BB, T, N, N), lambda i: (i, 0, 0, 0)),
            pl.BlockSpec((BB, N, IN_C), lambda i: (i, 0, 0)),
            pl.BlockSpec((IN_C, F_), lambda i: (0, 0)),
            pl.BlockSpec((1, F_), lambda i: (0, 0)),
            pl.BlockSpec((F_, F_), lambda i: (0, 0)),
            pl.BlockSpec((1, F_), lambda i: (0, 0)),
            pl.BlockSpec((1, F_), lambda i: (0, 0)),
            pl.BlockSpec((1, 1), lambda i: (0, 0)),
            pl.BlockSpec((N, NC), lambda i: (0, 0)),
            pl.BlockSpec((1, NC), lambda i: (0, 0)),
        ],
        out_specs=pl.BlockSpec((BB, NC), lambda i: (i, 0)),
        out_shape=jax.ShapeDtypeStruct((B, NC), jnp.float32),
    )(graph, real, W1, b1.reshape(1, F_), W2, b2.reshape(1, F_),
      Wlin.reshape(1, F_), blin.reshape(1, 1), Wconv.T, bconv.reshape(1, NC))
    return out


# speedup vs baseline: 97.2744x; 1.5787x over previous
import functools

import jax
import jax.numpy as jnp
from jax.experimental import pallas as pl

B, N, IN_C, F_, T, NC = 1024, 30, 128, 64, 16, 4
BB = 16


def _bmm_t(M, u):
    return jax.lax.dot_general(
        M, u, (((1,), (1,)), ((0,), (0,))),
        preferred_element_type=jnp.float32)


def _gcn_body(graph_ref, real_ref, W1_ref, b1_ref, W2_ref, b2_ref,
              Wlin_ref, blin_ref, WconvT_ref, bconv_ref, out_ref):
    g = graph_ref[...]
    adj = jnp.sum(g, axis=1) * (1.0 / T)
    w = (adj != 0.0).astype(jnp.float32)
    rr = jax.lax.broadcasted_iota(jnp.int32, (N, N), 0)
    cc = jax.lax.broadcasted_iota(jnp.int32, (N, N), 1)
    eye = (rr == cc).astype(jnp.float32)
    A = w + eye[None]
    deg = jnp.sum(A, axis=1)
    dinv = jax.lax.rsqrt(deg)
    M = dinv[:, :, None] * A * dinv[:, None, :]

    x = real_ref[...]
    b1 = b1_ref[...]
    b2 = b2_ref[...]

    h = jax.lax.dot_general(
        x, W1_ref[...], (((2,), (0,)), ((), ())),
        preferred_element_type=jnp.float32)
    h1 = jnp.maximum(_bmm_t(M, h) + b1[None], 0.0)
    g2 = jax.lax.dot_general(
        h1, W2_ref[...], (((2,), (0,)), ((), ())),
        preferred_element_type=jnp.float32)
    h2a = jnp.maximum(_bmm_t(M, g2) + b2[None], 0.0)

    lin = jnp.sum(h2a * Wlin_ref[...][None], axis=2)
    xl = jnp.maximum(lin + blin_ref[0, 0], 0.0)
    out = jnp.dot(xl, WconvT_ref[...],
                  preferred_element_type=jnp.float32) + bconv_ref[...]
    out_ref[...] = out


@jax.jit
def kernel(real, imag, graph, W1, b1, W2, b2, Wlin, blin, Wconv, bconv):
    del imag
    grid = (B // BB,)
    out = pl.pallas_call(
        _gcn_body,
        grid=grid,
        in_specs=[
            pl.BlockSpec((BB, T, N, N), lambda i: (i, 0, 0, 0)),
            pl.BlockSpec((BB, N, IN_C), lambda i: (i, 0, 0)),
            pl.BlockSpec((IN_C, F_), lambda i: (0, 0)),
            pl.BlockSpec((1, F_), lambda i: (0, 0)),
            pl.BlockSpec((F_, F_), lambda i: (0, 0)),
            pl.BlockSpec((1, F_), lambda i: (0, 0)),
            pl.BlockSpec((1, F_), lambda i: (0, 0)),
            pl.BlockSpec((1, 1), lambda i: (0, 0)),
            pl.BlockSpec((N, NC), lambda i: (0, 0)),
            pl.BlockSpec((1, NC), lambda i: (0, 0)),
        ],
        out_specs=pl.BlockSpec((BB, NC), lambda i: (i, 0)),
        out_shape=jax.ShapeDtypeStruct((B, NC), jnp.float32),
    )(graph, real, W1, b1.reshape(1, F_), W2, b2.reshape(1, F_),
      Wlin.reshape(1, F_), blin.reshape(1, 1), Wconv.T, bconv.reshape(1, NC))
    return out
